# Initial kernel scaffold; baseline (speedup 1.0000x reference)
import jax
import jax.numpy as jnp
from jax.experimental import pallas as pl


def kernel(task_token_ids, embed_table, pos):
    raise NotImplementedError("write your pallas kernel here")


# speedup vs baseline: 2.1235x; 2.1235x over previous
import functools

import jax
import jax.numpy as jnp
from jax import lax
from jax.experimental import pallas as pl
from jax.experimental.pallas import tpu as pltpu
from jax.experimental.pallas import tpu_sc as plsc

D = 128
SEQ = 200
BATCH = 4096
LANES = 16
NWORKERS = 32


def _sc_body(ids_hbm, pos_hbm, table_hbm, out_hbm,
             idx_v, rows_v, pos_v, sem):
    nc = 2
    wid = lax.axis_index("s") * nc + lax.axis_index("c")
    rows_per_w = BATCH // NWORKERS

    pltpu.sync_copy(pos_hbm, pos_v)

    def chunk_body(i, carry):
        base = (wid * rows_per_w + i) * SEQ
        pltpu.sync_copy(ids_hbm.at[pl.ds(base, SEQ)], idx_v)
        pltpu.async_copy(table_hbm.at[idx_v], rows_v, sem).wait()

        def add_body(r, carry2):
            for c in range(D // LANES):
                sl = pl.ds(c * LANES, LANES)
                rows_v[r, sl] = rows_v[r, sl] + pos_v[r, sl]
            return carry2

        lax.fori_loop(0, SEQ, add_body, 0, unroll=2)
        pltpu.sync_copy(rows_v, out_hbm.at[pl.ds(base, SEQ)])
        return carry

    lax.fori_loop(0, rows_per_w, chunk_body, 0)


def kernel(task_token_ids, embed_table, pos):
    b, l = task_token_ids.shape
    ids_flat = task_token_ids.reshape(-1)
    pos2 = pos.reshape(pos.shape[1], pos.shape[2])[:l]

    mesh = plsc.VectorSubcoreMesh(core_axis_name="c", subcore_axis_name="s")
    run = functools.partial(
        pl.kernel,
        mesh=mesh,
        out_type=jax.ShapeDtypeStruct((b * l, D), jnp.float32),
        scratch_types=[
            pltpu.VMEM((SEQ,), jnp.int32),
            pltpu.VMEM((SEQ, D), jnp.float32),
            pltpu.VMEM((SEQ, D), jnp.float32),
            pltpu.SemaphoreType.DMA,
        ],
    )(_sc_body)

    out = run(ids_flat, pos2, embed_table)
    return out.reshape(b, l, D)


# speedup vs baseline: 7.5123x; 3.5377x over previous
import functools

import jax
import jax.numpy as jnp
from jax import lax
from jax.experimental import pallas as pl
from jax.experimental.pallas import tpu as pltpu
from jax.experimental.pallas import tpu_sc as plsc

D = 128
SEQ = 200
BATCH = 4096
LANES = 16
NWORKERS = 32
CHUNKS = BATCH // NWORKERS


def _add_pos(rows_v, pos_v):
    @plsc.parallel_loop(0, SEQ, step=2)
    def _(r):
        for rr in range(2):
            for c in range(D // LANES):
                sl = pl.ds(c * LANES, LANES)
                plsc.addupdate(rows_v.at[r + rr, sl], pos_v[r + rr, sl])


def _sc_body(ids_hbm, pos_hbm, table_hbm, out_hbm,
             idx_v, rows0, rows1, pos_v, gsem0, gsem1, wsem0, wsem1):
    nc = 2
    wid = lax.axis_index("s") * nc + lax.axis_index("c")
    wbase = wid * CHUNKS * SEQ

    pltpu.sync_copy(pos_hbm, pos_v)
    pltpu.sync_copy(ids_hbm.at[pl.ds(wbase, CHUNKS * SEQ)], idx_v)

    rows = (rows0, rows1)
    gsem = (gsem0, gsem1)
    wsem = (wsem0, wsem1)

    def gather(g, p):
        pltpu.async_copy(table_hbm.at[idx_v.at[pl.ds(g * SEQ, SEQ)]],
                         rows[p], gsem[p])

    def gather_wait(p):
        pltpu.make_async_copy(table_hbm.at[idx_v.at[pl.ds(0, SEQ)]],
                              rows[p], gsem[p]).wait()

    def write(g, p):
        pltpu.async_copy(rows[p], out_hbm.at[pl.ds(wbase + g * SEQ, SEQ)],
                         wsem[p])

    def write_wait(p):
        pltpu.make_async_copy(rows[p], out_hbm.at[pl.ds(wbase, SEQ)],
                              wsem[p]).wait()

    gather(0, 0)
    gather(1, 1)
    gather_wait(0)
    _add_pos(rows0, pos_v)
    write(0, 0)

    def pair_body(k, carry):
        g = 2 + 2 * k
        write_wait(0)
        gather(g, 0)
        gather_wait(1)
        _add_pos(rows1, pos_v)
        write(g - 1, 1)
        write_wait(1)
        gather(g + 1, 1)
        gather_wait(0)
        _add_pos(rows0, pos_v)
        write(g, 0)
        return carry

    lax.fori_loop(0, (CHUNKS - 2) // 2, pair_body, 0)

    gather_wait(1)
    _add_pos(rows1, pos_v)
    write(CHUNKS - 1, 1)
    write_wait(0)
    write_wait(1)


def kernel(task_token_ids, embed_table, pos):
    b, l = task_token_ids.shape
    ids_flat = task_token_ids.reshape(-1)
    pos2 = pos.reshape(pos.shape[1], pos.shape[2])[:l]

    mesh = plsc.VectorSubcoreMesh(core_axis_name="c", subcore_axis_name="s")
    run = functools.partial(
        pl.kernel,
        mesh=mesh,
        out_type=jax.ShapeDtypeStruct((b * l, D), jnp.float32),
        scratch_types=[
            pltpu.VMEM((CHUNKS * SEQ,), jnp.int32),
            pltpu.VMEM((SEQ, D), jnp.float32),
            pltpu.VMEM((SEQ, D), jnp.float32),
            pltpu.VMEM((SEQ, D), jnp.float32),
            pltpu.SemaphoreType.DMA,
            pltpu.SemaphoreType.DMA,
            pltpu.SemaphoreType.DMA,
            pltpu.SemaphoreType.DMA,
        ],
    )(_sc_body)

    out = run(ids_flat, pos2, embed_table)
    return out.reshape(b, l, D)
